# baseline (speedup 1.0000x reference)
import jax
import jax.numpy as jnp
from jax import lax
from jax.experimental import pallas as pl
from jax.experimental.pallas import tpu as pltpu
from jax.experimental.pallas import tpu_sc as plsc

_NC = 2
_NS = 16
_NW = _NC * _NS
_R_BLK = 512
_S_CUT = 1024
_CH_ROWS = 16


def _tc_add_kernel(in_ref, emb_ref, out_ref):
    i = pl.program_id(0)
    seq_blocks = emb_ref.shape[0] // _R_BLK
    e0 = pl.multiple_of((i % seq_blocks) * _R_BLK, _R_BLK)
    out_ref[...] = in_ref[...] + emb_ref[pl.ds(e0, _R_BLK), :]


def _sc_body(in_hbm, emb_hbm, out_hbm,
             vin0, vin1, vemb0, vemb1, vout0, vout1,
             sin0, sin1, semb0, semb1, sout0, sout1):
    batch, seq_len, _ = in_hbm.shape
    sc_rows = seq_len - _S_CUT
    rows_per_w = sc_rows // _NW
    n_chunks = rows_per_w // _CH_ROWS

    wid = lax.axis_index("s") * _NC + lax.axis_index("c")
    bb = batch - 1
    s_base = _S_CUT + wid * rows_per_w

    vin = (vin0, vin1)
    vemb = (vemb0, vemb1)
    vout = (vout0, vout1)
    sin = (sin0, sin1)
    semb = (semb0, semb1)
    sout = (sout0, sout1)

    def start_in(c):
        b = c & 1
        s0 = s_base + c * _CH_ROWS
        d_in = pltpu.async_copy(
            in_hbm.at[bb, pl.ds(s0, _CH_ROWS), :], vin[b], sin[b])
        d_emb = pltpu.async_copy(
            emb_hbm.at[pl.ds(s0, _CH_ROWS), :], vemb[b], semb[b])
        return d_in, d_emb

    in_descs = {c: start_in(c) for c in range(min(2, n_chunks))}
    out_descs = {}

    for c in range(n_chunks):
        b = c & 1
        d_in, d_emb = in_descs.pop(c)
        d_in.wait()
        d_emb.wait()
        if c >= 2:
            out_descs.pop(c - 2).wait()

        @plsc.parallel_loop(0, _CH_ROWS * 1024, step=16, unroll=8)
        def _(i):
            r = i >> 10
            col = pl.multiple_of(i & 1023, 16)
            vout[b][r, pl.ds(col, 16)] = (
                vin[b][r, pl.ds(col, 16)] + vemb[b][r, pl.ds(col, 16)])

        o0 = (wid * rows_per_w) + c * _CH_ROWS
        out_descs[c] = pltpu.async_copy(
            vout[b], out_hbm.at[pl.ds(o0, _CH_ROWS), :], sout[b])
        if c + 2 < n_chunks:
            in_descs[c + 2] = start_in(c + 2)

    for c in sorted(out_descs):
        out_descs.pop(c).wait()


def kernel(inputs, embeddings):
    batch, seq_len, dim = inputs.shape
    pos = embeddings[:seq_len]
    flat = inputs.reshape(batch * seq_len, dim)
    tc_rows = (batch - 1) * seq_len + _S_CUT
    sc_rows = seq_len - _S_CUT

    mesh = plsc.VectorSubcoreMesh(
        core_axis_name="c", subcore_axis_name="s",
        num_cores=_NC, num_subcores=_NS)
    sc_run = pl.kernel(
        _sc_body,
        out_type=jax.ShapeDtypeStruct((sc_rows, dim), jnp.float32),
        mesh=mesh,
        compiler_params=pltpu.CompilerParams(use_tc_tiling_on_sc=True),
        scratch_types=[
            pltpu.VMEM((_CH_ROWS, dim), jnp.float32),
            pltpu.VMEM((_CH_ROWS, dim), jnp.float32),
            pltpu.VMEM((_CH_ROWS, dim), jnp.float32),
            pltpu.VMEM((_CH_ROWS, dim), jnp.float32),
            pltpu.VMEM((_CH_ROWS, dim), jnp.float32),
            pltpu.VMEM((_CH_ROWS, dim), jnp.float32),
            pltpu.SemaphoreType.DMA,
            pltpu.SemaphoreType.DMA,
            pltpu.SemaphoreType.DMA,
            pltpu.SemaphoreType.DMA,
            pltpu.SemaphoreType.DMA,
            pltpu.SemaphoreType.DMA,
        ],
    )
    sc_part = sc_run(inputs, pos)

    tc_out = pl.pallas_call(
        _tc_add_kernel,
        grid=(tc_rows // _R_BLK,),
        in_specs=[
            pl.BlockSpec((_R_BLK, dim), lambda i: (i, 0)),
            pl.BlockSpec((seq_len, dim), lambda i: (0, 0)),
        ],
        out_specs=pl.BlockSpec((_R_BLK, dim), lambda i: (i, 0)),
        out_shape=jax.ShapeDtypeStruct((batch * seq_len, dim), jnp.float32),
    )(flat, pos)

    out = lax.dynamic_update_slice(tc_out, sc_part, (tc_rows, 0))
    return out.reshape(batch, seq_len, dim)


# speedup vs baseline: 2.0426x; 2.0426x over previous
import jax
import jax.numpy as jnp
from jax.experimental import pallas as pl


_R_BLK = 2048


def _add_kernel(in_ref, emb_ref, out_ref):
    i = pl.program_id(0)
    seq_blocks = emb_ref.shape[0] // _R_BLK
    e0 = pl.multiple_of((i % seq_blocks) * _R_BLK, _R_BLK)
    out_ref[...] = in_ref[...] + emb_ref[pl.ds(e0, _R_BLK), :]


def kernel(inputs, embeddings):
    batch, seq_len, dim = inputs.shape
    pos = embeddings[:seq_len]
    flat = inputs.reshape(batch * seq_len, dim)
    grid = (batch * seq_len // _R_BLK,)
    out = pl.pallas_call(
        _add_kernel,
        grid=grid,
        in_specs=[
            pl.BlockSpec((_R_BLK, dim), lambda i: (i, 0)),
            pl.BlockSpec((seq_len, dim), lambda i: (0, 0)),
        ],
        out_specs=pl.BlockSpec((_R_BLK, dim), lambda i: (i, 0)),
        out_shape=jax.ShapeDtypeStruct((batch * seq_len, dim), inputs.dtype),
    )(flat, pos)
    return out.reshape(batch, seq_len, dim)
